# Initial kernel scaffold; baseline (speedup 1.0000x reference)
import jax
import jax.numpy as jnp
from jax.experimental import pallas as pl


def kernel(g, in_feat, W1, al1, ar1, b1, W2, al2, ar2, b2):
    raise NotImplementedError("write your pallas kernel here")


# speedup vs baseline: 21.3800x; 21.3800x over previous
import functools

import jax
import jax.numpy as jnp
from jax import lax
from jax.experimental import pallas as pl
from jax.experimental.pallas import tpu as pltpu
from jax.experimental.pallas import tpu_sc as plsc

_N = 10000
_E = 320000
_D = 128

_NC = 2
_NS = 16
_NW = _NC * _NS
_EPW = _E // _NW
_C = 80
_NCH = _EPW // _C
_RPT = _N // _NS

_ZR = 125


def _edge_body(ft_hbm, el_hbm, er_hbm, src_hbm, dst_hbm,
               acc_out, den_out,
               srcb, dstb, el_v, er_v, rows_v, w_v, zden,
               acc_sh, den_sh, sem):
    cid = lax.axis_index("c")
    sid = lax.axis_index("s")
    wid = cid * _NS + sid

    pltpu.sync_copy(el_hbm, el_v)
    pltpu.sync_copy(er_hbm, er_v)

    zv = jnp.zeros((16,), jnp.float32)

    def zrow(i, _):
        for k in range(_D // 16):
            rows_v[0, i, pl.ds(k * 16, 16)] = zv
        return _
    lax.fori_loop(0, _C, zrow, None)

    def zden_row(i, _):
        zden[pl.ds(i * 16, 16)] = zv
        return _
    lax.fori_loop(0, 40, zden_row, None)

    for j in range(7):
        pltpu.sync_copy(rows_v.at[0],
                        acc_sh.at[pl.ds(sid * _RPT + j * _C, _C)])
    pltpu.sync_copy(rows_v.at[0, pl.ds(0, _RPT - 7 * _C)],
                    acc_sh.at[pl.ds(sid * _RPT + 7 * _C, _RPT - 7 * _C)])

    @pl.when(sid < _NS - 1)
    def _():
        pltpu.sync_copy(zden.at[pl.ds(0, 624)],
                        den_sh.at[pl.ds(sid * 624, 624)])

    @pl.when(sid == _NS - 1)
    def _():
        pltpu.sync_copy(zden, den_sh.at[pl.ds(624 * (_NS - 1), 640)])

    plsc.subcore_barrier()

    def chunk(ci, _):
        pltpu.sync_copy(src_hbm.at[wid, ci], srcb.at[0])
        pltpu.sync_copy(dst_hbm.at[wid, ci], dstb.at[0])
        gather = pltpu.async_copy(ft_hbm.at[srcb.at[0]], rows_v.at[0], sem)

        def grp(gi, _):
            s16 = srcb[0, pl.ds(gi * 16, 16)]
            d16 = dstb[0, pl.ds(gi * 16, 16)]
            e = plsc.load_gather(el_v, [s16]) + plsc.load_gather(er_v, [d16])
            e = jnp.where(e >= 0.0, e, e * 0.2)
            w_v[0, pl.ds(gi * 16, 16)] = jnp.exp(e)
            return _
        lax.fori_loop(0, _C // 16, grp, None)

        gather.wait()

        def scale(gi, _):
            for j in range(16):
                ei = gi * 16 + j
                wb = plsc.load_gather(w_v.at[0],
                                      [jnp.full((16,), ei, jnp.int32)])
                for k in range(_D // 16):
                    rows_v[0, ei, pl.ds(k * 16, 16)] = (
                        rows_v[0, ei, pl.ds(k * 16, 16)] * wb)
            return _
        lax.fori_loop(0, _C // 16, scale, None)

        pltpu.sync_copy(rows_v.at[0], acc_sh.at[dstb.at[0]], add=True)
        pltpu.sync_copy(w_v.at[0], den_sh.at[dstb.at[0]], add=True)
        return _
    lax.fori_loop(0, _NCH, chunk, None)

    plsc.subcore_barrier()

    pltpu.sync_copy(acc_sh.at[pl.ds(sid * _RPT, _RPT)],
                    acc_out.at[cid, pl.ds(sid * _RPT, _RPT)])

    @pl.when(sid == 0)
    def _():
        pltpu.sync_copy(den_sh, den_out.at[cid])


def _edge_call(ft, el, er, src, dst):
    mesh = plsc.VectorSubcoreMesh(core_axis_name="c", subcore_axis_name="s",
                                  num_cores=_NC, num_subcores=_NS)
    f = pl.kernel(
        _edge_body,
        out_type=(jax.ShapeDtypeStruct((_NC, _N, _D), jnp.float32),
                  jax.ShapeDtypeStruct((_NC, _N), jnp.float32)),
        mesh=mesh,
        scratch_types=[
            pltpu.VMEM((2, _C), jnp.int32),
            pltpu.VMEM((2, _C), jnp.int32),
            pltpu.VMEM((_N,), jnp.float32),
            pltpu.VMEM((_N,), jnp.float32),
            pltpu.VMEM((2, _C, _D), jnp.float32),
            pltpu.VMEM((2, _C), jnp.float32),
            pltpu.VMEM((640,), jnp.float32),
            pltpu.VMEM_SHARED((_N, _D), jnp.float32),
            pltpu.VMEM_SHARED((_N,), jnp.float32),
            pltpu.SemaphoreType.DMA,
        ],
        compiler_params=pltpu.CompilerParams(use_tc_tiling_on_sc=False,
                                             needs_layout_passes=False),
    )
    return f(ft, el, er, src, dst)


_B = 2000


def _prep1_body(x_ref, w_ref, al_ref, ar_ref, ft_ref, el_ref, er_ref):
    ft = jnp.dot(x_ref[...], w_ref[...], preferred_element_type=jnp.float32)
    ft_ref[...] = ft
    el_ref[...] = jnp.sum(ft * al_ref[...], axis=1, keepdims=True)
    er_ref[...] = jnp.sum(ft * ar_ref[...], axis=1, keepdims=True)


def _prep1(x, W, al, ar):
    return pl.pallas_call(
        _prep1_body,
        grid=(_N // _B,),
        in_specs=[
            pl.BlockSpec((_B, _D), lambda i: (i, 0)),
            pl.BlockSpec((_D, _D), lambda i: (0, 0)),
            pl.BlockSpec((1, _D), lambda i: (0, 0)),
            pl.BlockSpec((1, _D), lambda i: (0, 0)),
        ],
        out_specs=[
            pl.BlockSpec((_B, _D), lambda i: (i, 0)),
            pl.BlockSpec((_B, 1), lambda i: (i, 0)),
            pl.BlockSpec((_B, 1), lambda i: (i, 0)),
        ],
        out_shape=[
            jax.ShapeDtypeStruct((_N, _D), jnp.float32),
            jax.ShapeDtypeStruct((_N, 1), jnp.float32),
            jax.ShapeDtypeStruct((_N, 1), jnp.float32),
        ],
    )(x, W, al, ar)


def _prep2_body(acc_ref, den_ref, b_ref, w_ref, al_ref, ar_ref,
                ft_ref, el_ref, er_ref):
    a = acc_ref[0] + acc_ref[1]
    dn = den_ref[0] + den_ref[1]
    h = a / (dn + 1e-9) + b_ref[...]
    ft = jnp.dot(h, w_ref[...], preferred_element_type=jnp.float32)
    ft_ref[...] = ft
    el_ref[...] = jnp.sum(ft * al_ref[...], axis=1, keepdims=True)
    er_ref[...] = jnp.sum(ft * ar_ref[...], axis=1, keepdims=True)


def _prep2(acc, den, b, W, al, ar):
    den = den.reshape(2, _N, 1)
    return pl.pallas_call(
        _prep2_body,
        grid=(_N // _B,),
        in_specs=[
            pl.BlockSpec((2, _B, _D), lambda i: (0, i, 0)),
            pl.BlockSpec((2, _B, 1), lambda i: (0, i, 0)),
            pl.BlockSpec((1, _D), lambda i: (0, 0)),
            pl.BlockSpec((_D, _D), lambda i: (0, 0)),
            pl.BlockSpec((1, _D), lambda i: (0, 0)),
            pl.BlockSpec((1, _D), lambda i: (0, 0)),
        ],
        out_specs=[
            pl.BlockSpec((_B, _D), lambda i: (i, 0)),
            pl.BlockSpec((_B, 1), lambda i: (i, 0)),
            pl.BlockSpec((_B, 1), lambda i: (i, 0)),
        ],
        out_shape=[
            jax.ShapeDtypeStruct((_N, _D), jnp.float32),
            jax.ShapeDtypeStruct((_N, 1), jnp.float32),
            jax.ShapeDtypeStruct((_N, 1), jnp.float32),
        ],
    )(acc, den, b, W, al, ar)


def _fin_body(acc_ref, den_ref, b_ref, out_ref):
    a = acc_ref[0] + acc_ref[1]
    dn = den_ref[0] + den_ref[1]
    out_ref[...] = a / (dn + 1e-9) + b_ref[...]


def _fin(acc, den, b):
    den = den.reshape(2, _N, 1)
    return pl.pallas_call(
        _fin_body,
        grid=(_N // _B,),
        in_specs=[
            pl.BlockSpec((2, _B, _D), lambda i: (0, i, 0)),
            pl.BlockSpec((2, _B, 1), lambda i: (0, i, 0)),
            pl.BlockSpec((1, _D), lambda i: (0, 0)),
        ],
        out_specs=pl.BlockSpec((_B, _D), lambda i: (i, 0)),
        out_shape=jax.ShapeDtypeStruct((_N, _D), jnp.float32),
    )(acc, den, b)


def kernel(g, in_feat, W1, al1, ar1, b1, W2, al2, ar2, b2):
    g = g.astype(jnp.int32)
    src = g[0].reshape(_NW, _NCH, _C)
    dst = g[1].reshape(_NW, _NCH, _C)
    b1r = b1.reshape(1, _D)
    b2r = b2.reshape(1, _D)

    ft1, el1, er1 = _prep1(in_feat, W1, al1, ar1)
    acc1, den1 = _edge_call(ft1, el1.reshape(_N), er1.reshape(_N), src, dst)
    ft2, el2, er2 = _prep2(acc1, den1, b1r, W2, al2, ar2)
    acc2, den2 = _edge_call(ft2, el2.reshape(_N), er2.reshape(_N), src, dst)
    out = _fin(acc2, den2, b2r)
    return out.reshape(_N, 1, _D)
